# baseline (speedup 1.0000x reference)
import functools

import jax
import jax.numpy as jnp
from jax import lax
from jax.experimental import pallas as pl
from jax.experimental.pallas import tpu as pltpu
from jax.experimental.pallas import tpu_sc as plsc

T_ROWS = 8192
D_COLS = 1024
NUM_CORES = 2
NUM_SUBCORES = 16
NUM_WORKERS = NUM_CORES * NUM_SUBCORES
ROWS_PER_WORKER = T_ROWS // NUM_WORKERS
CHUNK = 32
NCHUNK = ROWS_PER_WORKER // CHUNK
NBUF = 3


def _gather_body(idx_hbm, table_hbm, out_hbm, idx_v, buf, in_sem, out_sem):
    wid = lax.axis_index("s") * NUM_CORES + lax.axis_index("c")
    base = wid * ROWS_PER_WORKER

    pltpu.sync_copy(idx_hbm.at[pl.ds(base, ROWS_PER_WORKER)], idx_v)

    def gather(k, b):
        return pltpu.make_async_copy(
            table_hbm.at[idx_v.at[pl.ds(k * CHUNK, CHUNK)]], buf.at[b],
            in_sem.at[b])

    def put(k, b):
        return pltpu.make_async_copy(
            buf.at[b],
            out_hbm.at[pl.ds(base + k * CHUNK, CHUNK), 0],
            out_sem.at[b])

    gather(0, 0).start()
    for k in range(NCHUNK):
        b = k % NBUF
        gather(k, b).wait()
        if k + 1 < NCHUNK:
            nb = (k + 1) % NBUF
            if k + 1 >= NBUF:
                put(k + 1 - NBUF, nb).wait()
            gather(k + 1, nb).start()
        put(k, b).start()
    for k in range(max(0, NCHUNK - NBUF), NCHUNK):
        put(k, k % NBUF).wait()


@jax.jit
def _sc_gather(idx, table):
    mesh = plsc.VectorSubcoreMesh(core_axis_name="c", subcore_axis_name="s")
    return pl.kernel(
        _gather_body,
        mesh=mesh,
        out_type=jax.ShapeDtypeStruct((T_ROWS, 1, D_COLS), jnp.float32),
        scratch_types=[
            pltpu.VMEM((ROWS_PER_WORKER,), jnp.int32),
            pltpu.VMEM((NBUF, CHUNK, D_COLS), jnp.float32),
            pltpu.SemaphoreType.DMA((NBUF,)),
            pltpu.SemaphoreType.DMA((NBUF,)),
        ],
    )(idx, table)


def kernel(TDist, table):
    return _sc_gather(jnp.asarray(TDist, jnp.int32).reshape(T_ROWS), table)


# speedup vs baseline: 1.0645x; 1.0645x over previous
import functools

import jax
import jax.numpy as jnp
from jax import lax
from jax.experimental import pallas as pl
from jax.experimental.pallas import tpu as pltpu
from jax.experimental.pallas import tpu_sc as plsc

T_ROWS = 8192
D_COLS = 1024
NUM_CORES = 2
NUM_SUBCORES = 16
NUM_WORKERS = NUM_CORES * NUM_SUBCORES
ROWS_PER_WORKER = T_ROWS // NUM_WORKERS
CHUNK = 32
NCHUNK = ROWS_PER_WORKER // CHUNK
NBUF = 3


def _gather_body(idx_hbm, table_hbm, out_hbm, idx_v, buf, in_sem, out_sem):
    wid = lax.axis_index("s") * NUM_CORES + lax.axis_index("c")
    base = wid * ROWS_PER_WORKER

    pltpu.sync_copy(idx_hbm.at[pl.ds(base, ROWS_PER_WORKER)], idx_v)

    def gather(k, b):
        return pltpu.make_async_copy(
            table_hbm.at[idx_v.at[pl.ds(k * CHUNK, CHUNK)]], buf.at[b],
            in_sem.at[b])

    def put(k, b):
        return pltpu.make_async_copy(
            buf.at[b],
            out_hbm.at[pl.ds(base + k * CHUNK, CHUNK), 0],
            out_sem.at[b])

    gather(0, 0).start()
    gather(1, 1).start()
    for k in range(NCHUNK):
        b = k % NBUF
        gather(k, b).wait()
        nk = k + NBUF - 1
        if nk < NCHUNK:
            nb = nk % NBUF
            if k > 0:
                put(k - 1, nb).wait()
            gather(nk, nb).start()
        put(k, b).start()
    for k in range(max(0, NCHUNK - NBUF), NCHUNK):
        put(k, k % NBUF).wait()


@jax.jit
def _sc_gather(idx, table):
    mesh = plsc.VectorSubcoreMesh(core_axis_name="c", subcore_axis_name="s")
    return pl.kernel(
        _gather_body,
        mesh=mesh,
        out_type=jax.ShapeDtypeStruct((T_ROWS, 1, D_COLS), jnp.float32),
        scratch_types=[
            pltpu.VMEM((ROWS_PER_WORKER,), jnp.int32),
            pltpu.VMEM((NBUF, CHUNK, D_COLS), jnp.float32),
            pltpu.SemaphoreType.DMA((NBUF,)),
            pltpu.SemaphoreType.DMA((NBUF,)),
        ],
    )(idx, table)


def kernel(TDist, table):
    return _sc_gather(jnp.asarray(TDist, jnp.int32).reshape(T_ROWS), table)
